# Initial kernel scaffold; baseline (speedup 1.0000x reference)
import jax
import jax.numpy as jnp
from jax.experimental import pallas as pl


def kernel(x, y, lengths):
    raise NotImplementedError("write your pallas kernel here")


# speedup vs baseline: 1.2268x; 1.2268x over previous
import functools

import jax
import jax.numpy as jnp
from jax import lax
from jax.experimental import pallas as pl
from jax.experimental.pallas import tpu as pltpu
from jax.experimental.pallas import tpu_sc as plsc

_L = 16
_SEQ = 2048
_ROWS = 2
_STEPS = _SEQ // _L


def _sc_sample_loss(x, y2, lens):
    mesh = plsc.VectorSubcoreMesh(core_axis_name="c", subcore_axis_name="s")

    @functools.partial(
        pl.kernel,
        out_type=jax.ShapeDtypeStruct((1,), jnp.float32),
        mesh=mesh,
        compiler_params=pltpu.CompilerParams(needs_layout_passes=False),
        scratch_types=[
            pltpu.VMEM((_ROWS, 32768), jnp.float32),
            pltpu.VMEM((_ROWS, _SEQ), jnp.int32),
            pltpu.VMEM((_L,), jnp.int32),
            pltpu.VMEM((_L,), jnp.float32),
        ],
    )
    def k(x_hbm, y_hbm, len_hbm, out_hbm, x_v, idx_v, len_v, res_v):
        is_worker = jnp.logical_and(
            lax.axis_index("c") == 0, lax.axis_index("s") == 0
        )

        @pl.when(is_worker)
        def _():
            for i in range(_ROWS):
                pltpu.sync_copy(x_hbm.at[i], x_v.at[i])
                pltpu.sync_copy(y_hbm.at[i], idx_v.at[i])
            pltpu.sync_copy(len_hbm, len_v)

            lane = lax.iota(jnp.int32, _L)
            lv = len_v[...]
            total = jnp.zeros((_L,), jnp.float32)
            for i in range(_ROWS):
                row_sel = jnp.full((_L,), i, dtype=jnp.int32)
                li = jnp.zeros((_L,), dtype=jnp.int32) + lv[i]

                def body(j, acc, i=i, row_sel=row_sel, li=li, lane=lane):
                    col = idx_v[i, pl.ds(j * _L, _L)]
                    vals = plsc.load_gather(x_v, [row_sel, col])
                    pos = lane + j * _L
                    keep = pos < li
                    return acc * jnp.where(keep, 1.0 - vals, 1.0)

                acc = lax.fori_loop(
                    0, _STEPS, body, jnp.ones((_L,), jnp.float32)
                )
                for sh in (8, 4, 2, 1):
                    res_v[...] = acc
                    perm = (lane + sh) & (_L - 1)
                    acc = acc * plsc.load_gather(res_v, [perm])
                total = total + (1.0 - acc)

            res_v[...] = total
            pltpu.sync_copy(res_v.at[pl.ds(0, 1)], out_hbm)

    return k(x, y2, lens)


def kernel(x, y, lengths):
    y2 = y[:_ROWS].astype(jnp.int32)
    lens = lengths.astype(jnp.int32)
    return _sc_sample_loss(x, y2, lens)


# speedup vs baseline: 1.2724x; 1.0372x over previous
import functools

import jax
import jax.numpy as jnp
from jax import lax
from jax.experimental import pallas as pl
from jax.experimental.pallas import tpu as pltpu
from jax.experimental.pallas import tpu_sc as plsc

_L = 16
_SEQ = 2048
_ROWS = 2
_NSUB = 16
_CHUNK = _SEQ // _NSUB
_CSTEPS = _CHUNK // _L


def _sc_sample_loss(x_flat, y2, lens):
    mesh = plsc.VectorSubcoreMesh(core_axis_name="c", subcore_axis_name="s")

    @functools.partial(
        pl.kernel,
        out_type=jax.ShapeDtypeStruct((1,), jnp.float32),
        mesh=mesh,
        compiler_params=pltpu.CompilerParams(needs_layout_passes=False),
        scratch_types=[
            pltpu.VMEM((_ROWS, _CHUNK), jnp.int32),
            pltpu.VMEM((_ROWS, _CHUNK), jnp.float32),
            pltpu.VMEM((_L,), jnp.int32),
            pltpu.VMEM((_ROWS, _L), jnp.float32),
            pltpu.VMEM((_NSUB, _ROWS, _L), jnp.float32),
            pltpu.VMEM((_L,), jnp.float32),
            pltpu.VMEM_SHARED((_NSUB, _ROWS, _L), jnp.float32),
            pltpu.SemaphoreType.DMA,
        ],
    )
    def k(x_hbm, y_hbm, len_hbm, out_hbm,
          idx_v, val_v, len_v, part_v, gath_v, res_v, shared, sem):
        cid = lax.axis_index("c")
        sid = lax.axis_index("s")
        on_core0 = cid == 0

        @pl.when(on_core0)
        def _work():
            base = sid * _CHUNK
            pltpu.sync_copy(len_hbm, len_v)
            lv = len_v[...]
            lane = lax.iota(jnp.int32, _L)
            ones = jnp.ones((_L,), jnp.float32)
            for i in range(_ROWS):
                part_v[i, pl.ds(0, _L)] = ones

                @pl.when(lv[i] > base)
                def _row(i=i, lv=lv, base=base, lane=lane, ones=ones):
                    pltpu.sync_copy(
                        y_hbm.at[i, pl.ds(base, _CHUNK)], idx_v.at[i]
                    )
                    if i:
                        for j in range(_CSTEPS):
                            sl = pl.ds(j * _L, _L)
                            idx_v[i, sl] = idx_v[i, sl] + i * 32768
                    pltpu.async_copy(
                        x_hbm.at[idx_v.at[i]], val_v.at[i], sem
                    ).wait()
                    li = jnp.zeros((_L,), jnp.int32) + lv[i]
                    nsteps = jnp.minimum(
                        _CSTEPS, (lv[i] - base + _L - 1) // _L
                    )

                    def body(j, acc, i=i, li=li, base=base, lane=lane):
                        v = val_v[i, pl.ds(j * _L, _L)]
                        pos = lane + base + j * _L
                        return acc * jnp.where(pos < li, 1.0 - v, 1.0)

                    part_v[i, pl.ds(0, _L)] = lax.fori_loop(
                        0, nsteps, body, ones
                    )

            pltpu.sync_copy(part_v, shared.at[sid])

        plsc.subcore_barrier()

        @pl.when(jnp.logical_and(on_core0, sid == 0))
        def _combine():
            pltpu.sync_copy(shared, gath_v)
            lane = lax.iota(jnp.int32, _L)
            total = jnp.zeros((_L,), jnp.float32)
            for i in range(_ROWS):
                acc = gath_v[0, i, pl.ds(0, _L)]
                for t in range(1, _NSUB):
                    acc = acc * gath_v[t, i, pl.ds(0, _L)]
                for sh in (8, 4, 2, 1):
                    res_v[...] = acc
                    perm = (lane + sh) & (_L - 1)
                    acc = acc * plsc.load_gather(res_v, [perm])
                total = total + (1.0 - acc)
            res_v[...] = total
            pltpu.sync_copy(res_v.at[pl.ds(0, 1)], out_hbm)

    return k(x_flat, y2, lens)


def kernel(x, y, lengths):
    x_flat = x.reshape(-1)
    y2 = y[:_ROWS].astype(jnp.int32)
    lens = lengths.astype(jnp.int32)
    return _sc_sample_loss(x_flat, y2, lens)
